# baseline (speedup 1.0000x reference)
import functools
import math

import jax
import jax.numpy as jnp
from jax import lax
from jax.experimental import pallas as pl
from jax.experimental.pallas import tpu as pltpu
from jax.experimental.pallas import tpu_sc as plsc

DIM = 32
BATCH = 16384
NUM_FIELDS = 5
SCALE = 1.0 / math.sqrt(NUM_FIELDS)

NC = 2
NS = 16
NW = NC * NS
BPW = BATCH // NW
CHUNK = 128
NCH = BPW // CHUNK


def _sc_gather_sum(i0, i1, i2, i3, i4, t0, t1, t2, t3, t4):
    mesh = plsc.VectorSubcoreMesh(core_axis_name="c", subcore_axis_name="s")

    @functools.partial(
        pl.kernel,
        mesh=mesh,
        out_type=jax.ShapeDtypeStruct((BATCH, DIM), jnp.float32),
        scratch_types=(
            [pltpu.VMEM((NCH, CHUNK), jnp.int32) for _ in range(NUM_FIELDS)]
            + [pltpu.VMEM((BPW, DIM), jnp.float32) for _ in range(NUM_FIELDS)]
            + [pltpu.SemaphoreType.DMA]
        ),
        compiler_params=pltpu.CompilerParams(use_tc_tiling_on_sc=False),
    )
    def body(i0h, i1h, i2h, i3h, i4h, t0h, t1h, t2h, t3h, t4h, out_h,
             x0, x1, x2, x3, x4, r0, r1, r2, r3, r4, sem):
        wid = lax.axis_index("s") * NC + lax.axis_index("c")
        idx_refs = (x0, x1, x2, x3, x4)
        row_refs = (r0, r1, r2, r3, r4)
        for ih, xv in zip((i0h, i1h, i2h, i3h, i4h), idx_refs):
            pltpu.sync_copy(ih.at[wid], xv)
        copies = []
        for th, xv, rv in zip((t0h, t1h, t2h, t3h, t4h), idx_refs, row_refs):
            for j in range(NCH):
                copies.append(pltpu.async_copy(
                    th.at[xv.at[j]], rv.at[pl.ds(j * CHUNK, CHUNK)], sem))
        for c in copies:
            c.wait()

        def acc_row(r, carry):
            for h in range(DIM // 16):
                sl = pl.ds(h * 16, 16)
                v = (r0[r, sl] + r1[r, sl] + r2[r, sl] + r3[r, sl]
                     + r4[r, sl]) * SCALE
                r0[r, sl] = v
            return carry

        lax.fori_loop(0, BPW, acc_row, 0)
        pltpu.sync_copy(r0, out_h.at[pl.ds(wid * BPW, BPW)])

    resh = lambda ix: ix.reshape(NW, NCH, CHUNK)
    return body(resh(i0), resh(i1), resh(i2), resh(i3), resh(i4),
                t0, t1, t2, t3, t4)


def _gelu_exact(x):
    return 0.5 * x * (1.0 + lax.erf(x * (1.0 / math.sqrt(2.0))))


def _mlp_tc(x, w0, b0, w1, b1, w2, b2):
    blk = 2048

    def body(x_ref, w0_ref, b0_ref, w1_ref, b1_ref, w2_ref, b2_ref, o_ref):
        h = x_ref[...]
        h = _gelu_exact(jnp.dot(h, w0_ref[...],
                                preferred_element_type=jnp.float32) + b0_ref[...])
        h = _gelu_exact(jnp.dot(h, w1_ref[...],
                                preferred_element_type=jnp.float32) + b1_ref[...])
        o_ref[...] = jnp.dot(h, w2_ref[...],
                             preferred_element_type=jnp.float32) + b2_ref[...]

    wspec = pl.BlockSpec((DIM, DIM), lambda i: (0, 0))
    bspec = pl.BlockSpec((1, DIM), lambda i: (0, 0))
    return pl.pallas_call(
        body,
        grid=(BATCH // blk,),
        in_specs=[pl.BlockSpec((blk, DIM), lambda i: (i, 0)),
                  wspec, bspec, wspec, bspec, wspec, bspec],
        out_specs=pl.BlockSpec((blk, DIM), lambda i: (i, 0)),
        out_shape=jax.ShapeDtypeStruct((BATCH, DIM), jnp.float32),
    )(x, w0, b0.reshape(1, DIM), w1, b1.reshape(1, DIM), w2, b2.reshape(1, DIM))


def kernel(idx_origin, idx_dest, idx_carrier, idx_tail_num, idx_flight_num,
           emb_origin, emb_dest, emb_carrier, emb_tail_num, emb_flight_num,
           W0, b0, W1, b1, W2, b2):
    x = _sc_gather_sum(idx_origin, idx_dest, idx_carrier, idx_tail_num,
                       idx_flight_num, emb_origin, emb_dest, emb_carrier,
                       emb_tail_num, emb_flight_num)
    return _mlp_tc(x, W0, b0, W1, b1, W2, b2)


# speedup vs baseline: 1.5903x; 1.5903x over previous
import functools
import math

import jax
import jax.numpy as jnp
from jax import lax
from jax.experimental import pallas as pl
from jax.experimental.pallas import tpu as pltpu
from jax.experimental.pallas import tpu_sc as plsc

DIM = 32
BATCH = 16384
NUM_FIELDS = 5
SCALE = 1.0 / math.sqrt(NUM_FIELDS)

NC = 2
NS = 16
NW = NC * NS
BPW = BATCH // NW
CHUNK = 128
NCH = BPW // CHUNK

V_TAIL = 1000000
TAIL_MAIN = (V_TAIL // 128) * 128
RING = 8


def _sc_tail_gather(idx_tail, tail_t):
    mesh = plsc.VectorSubcoreMesh(core_axis_name="c", subcore_axis_name="s")

    @functools.partial(
        pl.kernel,
        mesh=mesh,
        out_type=jax.ShapeDtypeStruct((BATCH, 128), jnp.float32),
        scratch_types=[
            pltpu.VMEM((BPW,), jnp.int32),
            pltpu.VMEM((RING, DIM, 128), jnp.float32),
            pltpu.VMEM((BPW, 128), jnp.float32),
            pltpu.SemaphoreType.DMA,
        ],
        compiler_params=pltpu.CompilerParams(use_tc_tiling_on_sc=True,
                                             needs_layout_passes=False),
    )
    def body(idx_h, tab_h, out_h, idx_v, blk, acc, sem):
        wid = lax.axis_index("s") * NC + lax.axis_index("c")
        base = wid * BPW
        pltpu.sync_copy(idx_h.at[wid], idx_v)
        rows16 = lax.iota(jnp.int32, 16)

        def fire(r, slot):
            c = jnp.where(r < TAIL_MAIN, r, 0) >> 7
            start = pl.multiple_of(c * 128, 128)
            pltpu.async_copy(tab_h.at[:, pl.ds(start, 128)], blk.at[slot], sem)

        def drain(slot):
            pltpu.make_async_copy(tab_h.at[:, pl.ds(0, 128)],
                                  blk.at[slot], sem).wait()

        def extract(b, r, slot):
            lane = jnp.where(r < TAIL_MAIN, r, 0) & 127
            w = jnp.where(r < TAIL_MAIN, SCALE, 0.0).astype(jnp.float32)
            lvec = jnp.full((16,), lane, jnp.int32)
            svec = jnp.full((16,), slot, jnp.int32)
            lo = plsc.load_gather(blk, [svec, rows16, lvec])
            hi = plsc.load_gather(blk, [svec, rows16 + 16, lvec])
            acc[b, pl.ds(0, 16)] = lo * w
            acc[b, pl.ds(16, 16)] = hi * w

        v0 = idx_v[pl.ds(0, 16)]
        for j in range(RING):
            fire(v0[j], j)

        def chunk_body(ci, _):
            v = idx_v[pl.ds(ci * 16, 16)]
            nxt = jnp.where(ci + 1 < BPW // 16, ci + 1, 0)
            vn = idx_v[pl.ds(nxt * 16, 16)]
            for j in range(16):
                b = ci * 16 + j
                sl = j % RING
                drain(sl)
                extract(b, v[j], sl)
                ahead = v[j + RING] if j + RING < 16 else vn[j + RING - 16]
                @pl.when(b + RING < BPW)
                def _fire_ahead(ahead=ahead, sl=sl):
                    fire(ahead, sl)
            return _

        lax.fori_loop(0, BPW // 16, chunk_body, 0)
        pltpu.sync_copy(acc, out_h.at[pl.ds(base, BPW)])

    return body(idx_tail.reshape(NW, BPW), tail_t)


def _sc_small_gather(i0, i1, i2, i3, i4, t0, t1, t2, t3, rem):
    mesh = plsc.VectorSubcoreMesh(core_axis_name="c", subcore_axis_name="s")

    @functools.partial(
        pl.kernel,
        mesh=mesh,
        out_type=jax.ShapeDtypeStruct((BATCH, DIM), jnp.float32),
        scratch_types=(
            [pltpu.VMEM((NCH, CHUNK), jnp.int32) for _ in range(NUM_FIELDS)]
            + [pltpu.VMEM((BPW, DIM), jnp.float32) for _ in range(NUM_FIELDS)]
            + [pltpu.SemaphoreType.DMA]
        ),
        compiler_params=pltpu.CompilerParams(use_tc_tiling_on_sc=False),
    )
    def body(i0h, i1h, i2h, i3h, i4h, t0h, t1h, t2h, t3h, remh, out_h,
             x0, x1, x2, x3, x4, r0, r1, r2, r3, r4, sem):
        wid = lax.axis_index("s") * NC + lax.axis_index("c")
        idx_refs = (x0, x1, x2, x3, x4)
        row_refs = (r0, r1, r2, r3, r4)
        for ih, xv in zip((i0h, i1h, i2h, i3h, i4h), idx_refs):
            pltpu.sync_copy(ih.at[wid], xv)
        copies = []
        for th, xv, rv in zip((t0h, t1h, t2h, t3h, remh), idx_refs, row_refs):
            for j in range(NCH):
                copies.append(pltpu.async_copy(
                    th.at[xv.at[j]], rv.at[pl.ds(j * CHUNK, CHUNK)], sem))
        for c in copies:
            c.wait()

        def acc_row(r, carry):
            for h in range(DIM // 16):
                sl = pl.ds(h * 16, 16)
                r0[r, sl] = (r0[r, sl] + r1[r, sl] + r2[r, sl] + r3[r, sl]
                             + r4[r, sl]) * SCALE
            return carry

        lax.fori_loop(0, BPW, acc_row, 0)
        pltpu.sync_copy(r0, out_h.at[pl.ds(wid * BPW, BPW)])

    resh = lambda ix: ix.reshape(NW, NCH, CHUNK)
    return body(resh(i0), resh(i1), resh(i2), resh(i3), resh(i4),
                t0, t1, t2, t3, rem)


def _gelu_exact(x):
    return 0.5 * x * (1.0 + lax.erf(x * (1.0 / math.sqrt(2.0))))


def _mlp_tc(xa, xb, w0, b0, w1, b1, w2, b2):
    blk = 2048

    def body(a_ref, b_ref, w0_ref, b0_ref, w1_ref, b1_ref, w2_ref, b2_ref,
             o_ref):
        h = a_ref[...] + b_ref[...]
        h = _gelu_exact(jnp.dot(h, w0_ref[...],
                                preferred_element_type=jnp.float32) + b0_ref[...])
        h = _gelu_exact(jnp.dot(h, w1_ref[...],
                                preferred_element_type=jnp.float32) + b1_ref[...])
        o_ref[...] = jnp.dot(h, w2_ref[...],
                             preferred_element_type=jnp.float32) + b2_ref[...]

    xspec = pl.BlockSpec((blk, DIM), lambda i: (i, 0))
    wspec = pl.BlockSpec((DIM, DIM), lambda i: (0, 0))
    bspec = pl.BlockSpec((1, DIM), lambda i: (0, 0))
    return pl.pallas_call(
        body,
        grid=(BATCH // blk,),
        in_specs=[xspec, xspec, wspec, bspec, wspec, bspec, wspec, bspec],
        out_specs=xspec,
        out_shape=jax.ShapeDtypeStruct((BATCH, DIM), jnp.float32),
    )(xa, xb, w0, b0.reshape(1, DIM), w1, b1.reshape(1, DIM),
      w2, b2.reshape(1, DIM))


def kernel(idx_origin, idx_dest, idx_carrier, idx_tail_num, idx_flight_num,
           emb_origin, emb_dest, emb_carrier, emb_tail_num, emb_flight_num,
           W0, b0, W1, b1, W2, b2):
    xt = _sc_tail_gather(idx_tail_num, emb_tail_num.T)[:, :DIM]
    rem = jnp.concatenate(
        [jnp.zeros((1, DIM), jnp.float32), emb_tail_num[TAIL_MAIN:, :],
         jnp.zeros((1024 - 1 - (V_TAIL - TAIL_MAIN), DIM), jnp.float32)],
        axis=0)
    idx_rem = jnp.where(idx_tail_num >= TAIL_MAIN,
                        idx_tail_num - TAIL_MAIN + 1, 0)
    xs = _sc_small_gather(idx_origin, idx_dest, idx_carrier, idx_flight_num,
                          idx_rem, emb_origin, emb_dest, emb_carrier,
                          emb_flight_num, rem)
    return _mlp_tc(xt, xs, W0, b0, W1, b1, W2, b2)


# speedup vs baseline: 2.8111x; 1.7677x over previous
import functools
import math

import jax
import jax.numpy as jnp
from jax import lax
from jax.experimental import pallas as pl
from jax.experimental.pallas import tpu as pltpu
from jax.experimental.pallas import tpu_sc as plsc

DIM = 32
BATCH = 16384
NUM_FIELDS = 5
SCALE = 1.0 / math.sqrt(NUM_FIELDS)

NC = 2
NS = 16
NW = NC * NS
BPW = BATCH // NW
CHUNK = 128
NCH = BPW // CHUNK

V_TAIL = 1000000
TAIL_MAIN = (V_TAIL // 128) * 128
RING = 8


def _sc_tail_gather(idx_tail, tail_t):
    mesh = plsc.VectorSubcoreMesh(core_axis_name="c", subcore_axis_name="s")

    @functools.partial(
        pl.kernel,
        mesh=mesh,
        out_type=jax.ShapeDtypeStruct((BATCH, 128), jnp.float32),
        scratch_types=[
            pltpu.VMEM((BPW,), jnp.int32),
            pltpu.VMEM((RING, DIM, 128), jnp.float32),
            pltpu.VMEM((BPW, 128), jnp.float32),
            pltpu.SemaphoreType.DMA,
        ],
        compiler_params=pltpu.CompilerParams(use_tc_tiling_on_sc=True,
                                             needs_layout_passes=False),
    )
    def body(idx_h, tab_h, out_h, idx_v, blk, acc, sem):
        wid = lax.axis_index("s") * NC + lax.axis_index("c")
        base = wid * BPW
        pltpu.sync_copy(idx_h.at[wid], idx_v)
        rows16 = lax.iota(jnp.int32, 16)

        def fire(r, slot):
            c = jnp.where(r < TAIL_MAIN, r, 0) >> 7
            start = pl.multiple_of(c * 128, 128)
            pltpu.async_copy(tab_h.at[:, pl.ds(start, 128)], blk.at[slot], sem)

        def drain(slot):
            pltpu.make_async_copy(tab_h.at[:, pl.ds(0, 128)],
                                  blk.at[slot], sem).wait()

        def extract(b, r, slot):
            lane = jnp.where(r < TAIL_MAIN, r, 0) & 127
            w = jnp.where(r < TAIL_MAIN, SCALE, 0.0).astype(jnp.float32)
            lvec = jnp.full((16,), lane, jnp.int32)
            svec = jnp.full((16,), slot, jnp.int32)
            lo = plsc.load_gather(blk, [svec, rows16, lvec])
            hi = plsc.load_gather(blk, [svec, rows16 + 16, lvec])
            acc[b, pl.ds(0, 16)] = lo * w
            acc[b, pl.ds(16, 16)] = hi * w

        v0 = idx_v[pl.ds(0, 16)]
        for j in range(RING):
            fire(v0[j], j)

        def chunk_body(ci, _):
            v = idx_v[pl.ds(ci * 16, 16)]
            nxt = jnp.where(ci + 1 < BPW // 16, ci + 1, 0)
            vn = idx_v[pl.ds(nxt * 16, 16)]
            for j in range(16):
                b = ci * 16 + j
                sl = j % RING
                drain(sl)
                extract(b, v[j], sl)
                ahead = v[j + RING] if j + RING < 16 else vn[j + RING - 16]
                @pl.when(b + RING < BPW)
                def _fire_ahead(ahead=ahead, sl=sl):
                    fire(ahead, sl)
            return _

        lax.fori_loop(0, BPW // 16, chunk_body, 0)
        pltpu.sync_copy(acc, out_h.at[pl.ds(base, BPW)])

    return body(idx_tail.reshape(NW, BPW), tail_t)


def _sc_small_gather(i0, i1, i2, i3, i4, t0, t1, t2, t3, rem):
    mesh = plsc.VectorSubcoreMesh(core_axis_name="c", subcore_axis_name="s")

    @functools.partial(
        pl.kernel,
        mesh=mesh,
        out_type=jax.ShapeDtypeStruct((BATCH, DIM), jnp.float32),
        scratch_types=(
            [pltpu.VMEM((NCH, CHUNK), jnp.int32) for _ in range(NUM_FIELDS)]
            + [pltpu.VMEM((BPW, DIM), jnp.float32) for _ in range(NUM_FIELDS)]
            + [pltpu.SemaphoreType.DMA]
        ),
        compiler_params=pltpu.CompilerParams(use_tc_tiling_on_sc=False),
    )
    def body(i0h, i1h, i2h, i3h, i4h, t0h, t1h, t2h, t3h, remh, out_h,
             x0, x1, x2, x3, x4, r0, r1, r2, r3, r4, sem):
        wid = lax.axis_index("s") * NC + lax.axis_index("c")
        idx_refs = (x0, x1, x2, x3, x4)
        row_refs = (r0, r1, r2, r3, r4)
        for ih, xv in zip((i0h, i1h, i2h, i3h, i4h), idx_refs):
            pltpu.sync_copy(ih.at[wid], xv)
        copies = []
        for th, xv, rv in zip((t0h, t1h, t2h, t3h, remh), idx_refs, row_refs):
            for j in range(NCH):
                copies.append(pltpu.async_copy(
                    th.at[xv.at[j]], rv.at[pl.ds(j * CHUNK, CHUNK)], sem))
        for c in copies:
            c.wait()

        def acc_row(r, carry):
            for h in range(DIM // 16):
                sl = pl.ds(h * 16, 16)
                r0[r, sl] = (r0[r, sl] + r1[r, sl] + r2[r, sl] + r3[r, sl]
                             + r4[r, sl]) * SCALE
            return carry

        lax.fori_loop(0, BPW, acc_row, 0)
        pltpu.sync_copy(r0, out_h.at[pl.ds(wid * BPW, BPW)])

    resh = lambda ix: ix.reshape(NW, NCH, CHUNK)
    return body(resh(i0), resh(i1), resh(i2), resh(i3), resh(i4),
                t0, t1, t2, t3, rem)


def _gelu_exact(x):
    return 0.5 * x * (1.0 + lax.erf(x * (1.0 / math.sqrt(2.0))))


def _mlp_tc(xa, xb, w0, b0, w1, b1, w2, b2):
    blk = 2048

    def body(a_ref, b_ref, w0_ref, b0_ref, w1_ref, b1_ref, w2_ref, b2_ref,
             o_ref):
        h = a_ref[...][:, :DIM] + b_ref[...]
        h = _gelu_exact(jnp.dot(h, w0_ref[...],
                                preferred_element_type=jnp.float32) + b0_ref[...])
        h = _gelu_exact(jnp.dot(h, w1_ref[...],
                                preferred_element_type=jnp.float32) + b1_ref[...])
        o_ref[...] = jnp.dot(h, w2_ref[...],
                             preferred_element_type=jnp.float32) + b2_ref[...]

    aspec = pl.BlockSpec((blk, 128), lambda i: (i, 0))
    xspec = pl.BlockSpec((blk, DIM), lambda i: (i, 0))
    wspec = pl.BlockSpec((DIM, DIM), lambda i: (0, 0))
    bspec = pl.BlockSpec((1, DIM), lambda i: (0, 0))
    return pl.pallas_call(
        body,
        grid=(BATCH // blk,),
        in_specs=[aspec, xspec, wspec, bspec, wspec, bspec, wspec, bspec],
        out_specs=xspec,
        out_shape=jax.ShapeDtypeStruct((BATCH, DIM), jnp.float32),
    )(xa, xb, w0, b0.reshape(1, DIM), w1, b1.reshape(1, DIM),
      w2, b2.reshape(1, DIM))


def kernel(idx_origin, idx_dest, idx_carrier, idx_tail_num, idx_flight_num,
           emb_origin, emb_dest, emb_carrier, emb_tail_num, emb_flight_num,
           W0, b0, W1, b1, W2, b2):
    xt = _sc_tail_gather(idx_tail_num, emb_tail_num.T)
    rem = jnp.concatenate(
        [jnp.zeros((1, DIM), jnp.float32), emb_tail_num[TAIL_MAIN:, :],
         jnp.zeros((1024 - 1 - (V_TAIL - TAIL_MAIN), DIM), jnp.float32)],
        axis=0)
    n_rem = V_TAIL - TAIL_MAIN
    spread = 1 + n_rem + jax.lax.rem(
        jax.lax.iota(jnp.int32, BATCH), jnp.int32(1024 - 1 - n_rem))
    idx_rem = jnp.where(idx_tail_num >= TAIL_MAIN,
                        idx_tail_num - TAIL_MAIN + 1, spread)
    xs = _sc_small_gather(idx_origin, idx_dest, idx_carrier, idx_flight_num,
                          idx_rem, emb_origin, emb_dest, emb_carrier,
                          emb_flight_num, rem)
    return _mlp_tc(xt, xs, W0, b0, W1, b1, W2, b2)


# speedup vs baseline: 2.8618x; 1.0180x over previous
import functools
import math

import jax
import jax.numpy as jnp
from jax import lax
from jax.experimental import pallas as pl
from jax.experimental.pallas import tpu as pltpu
from jax.experimental.pallas import tpu_sc as plsc

DIM = 32
BATCH = 16384
NUM_FIELDS = 5
SCALE = 1.0 / math.sqrt(NUM_FIELDS)

NC = 2
NS = 16
NW = NC * NS
BPW = BATCH // NW
CHUNK = 128
NCH = BPW // CHUNK

V_TAIL = 1000000
TAIL_MAIN = (V_TAIL // 128) * 128
RING = 12
N_REM = V_TAIL - TAIL_MAIN
REM_ROWS = 1024
N_PAD_ROWS = REM_ROWS - 1 - N_REM


def _sc_tail_gather(idx_tail, tail_t):
    mesh = plsc.VectorSubcoreMesh(core_axis_name="c", subcore_axis_name="s")

    @functools.partial(
        pl.kernel,
        mesh=mesh,
        out_type=jax.ShapeDtypeStruct((BATCH, 128), jnp.float32),
        scratch_types=[
            pltpu.VMEM((BPW,), jnp.int32),
            pltpu.VMEM((RING, DIM, 128), jnp.float32),
            pltpu.VMEM((BPW, 128), jnp.float32),
            pltpu.SemaphoreType.DMA,
        ],
        compiler_params=pltpu.CompilerParams(use_tc_tiling_on_sc=True,
                                             needs_layout_passes=False),
    )
    def body(idx_h, tab_h, out_h, idx_v, blk, acc, sem):
        wid = lax.axis_index("s") * NC + lax.axis_index("c")
        base = wid * BPW
        pltpu.sync_copy(idx_h.at[pl.ds(base, BPW)], idx_v)
        rows16 = lax.iota(jnp.int32, 16)

        def fire(r, slot):
            c = jnp.where(r < TAIL_MAIN, r, 0) >> 7
            start = pl.multiple_of(c * 128, 128)
            pltpu.async_copy(tab_h.at[:, pl.ds(start, 128)], blk.at[slot], sem)

        def drain(slot):
            pltpu.make_async_copy(tab_h.at[:, pl.ds(0, 128)],
                                  blk.at[slot], sem).wait()

        def extract(b, r, slot):
            lane = jnp.where(r < TAIL_MAIN, r, 0) & 127
            w = jnp.where(r < TAIL_MAIN, SCALE, 0.0).astype(jnp.float32)
            lvec = jnp.full((16,), lane, jnp.int32)
            svec = jnp.full((16,), slot, jnp.int32)
            lo = plsc.load_gather(blk, [svec, rows16, lvec])
            hi = plsc.load_gather(blk, [svec, rows16 + 16, lvec])
            acc[b, pl.ds(0, 16)] = lo * w
            acc[b, pl.ds(16, 16)] = hi * w

        v0 = idx_v[pl.ds(0, 16)]
        for j in range(RING):
            fire(v0[j], j)

        def chunk_body(ci, _):
            v = idx_v[pl.ds(ci * 16, 16)]
            nxt = jnp.where(ci + 1 < BPW // 16, ci + 1, 0)
            vn = idx_v[pl.ds(nxt * 16, 16)]
            for j in range(16):
                b = ci * 16 + j
                sl = b % RING
                drain(sl)
                extract(b, v[j], sl)
                ahead = v[j + RING] if j + RING < 16 else vn[j + RING - 16]
                @pl.when(b + RING < BPW)
                def _fire_ahead(ahead=ahead, b=b):
                    fire(ahead, (b + RING) % RING)
            return _

        lax.fori_loop(0, BPW // 16, chunk_body, 0)
        pltpu.sync_copy(acc, out_h.at[pl.ds(base, BPW)])

    return body(idx_tail, tail_t)


def _sc_small_gather(i0, i1, i2, i3, i4, t0, t1, t2, t3, rem):
    mesh = plsc.VectorSubcoreMesh(core_axis_name="c", subcore_axis_name="s")

    @functools.partial(
        pl.kernel,
        mesh=mesh,
        out_type=jax.ShapeDtypeStruct((BATCH, DIM), jnp.float32),
        scratch_types=(
            [pltpu.VMEM((NCH, CHUNK), jnp.int32) for _ in range(NUM_FIELDS)]
            + [pltpu.VMEM((BPW, DIM), jnp.float32) for _ in range(NUM_FIELDS)]
            + [pltpu.SemaphoreType.DMA]
        ),
        compiler_params=pltpu.CompilerParams(use_tc_tiling_on_sc=False),
    )
    def body(i0h, i1h, i2h, i3h, i4h, t0h, t1h, t2h, t3h, remh, out_h,
             x0, x1, x2, x3, x4, r0, r1, r2, r3, r4, sem):
        wid = lax.axis_index("s") * NC + lax.axis_index("c")
        base = wid * BPW
        idx_refs = (x0, x1, x2, x3, x4)
        row_refs = (r0, r1, r2, r3, r4)
        for ih, xv in zip((i0h, i1h, i2h, i3h, i4h), idx_refs):
            for j in range(NCH):
                pltpu.sync_copy(ih.at[pl.ds(base + j * CHUNK, CHUNK)],
                                xv.at[j])

        iota16 = lax.iota(jnp.int32, 16)

        def rem_body(k, carry):
            jrow = k // (CHUNK // 16)
            off = (k % (CHUNK // 16)) * 16
            pos = base + jrow * CHUNK + off + iota16
            v = x4[jrow, pl.ds(off, 16)]
            spread = 1 + N_REM + lax.rem(pos, jnp.int32(N_PAD_ROWS))
            x4[jrow, pl.ds(off, 16)] = jnp.where(
                v >= TAIL_MAIN, v - TAIL_MAIN + 1, spread)
            return carry

        lax.fori_loop(0, BPW // 16, rem_body, 0)

        copies = []
        for th, xv, rv in zip((t0h, t1h, t2h, t3h, remh), idx_refs, row_refs):
            for j in range(NCH):
                copies.append(pltpu.async_copy(
                    th.at[xv.at[j]], rv.at[pl.ds(j * CHUNK, CHUNK)], sem))
        for c in copies:
            c.wait()

        def acc_row(r, carry):
            for h in range(DIM // 16):
                sl = pl.ds(h * 16, 16)
                r0[r, sl] = (r0[r, sl] + r1[r, sl] + r2[r, sl] + r3[r, sl]
                             + r4[r, sl]) * SCALE
            return carry

        lax.fori_loop(0, BPW, acc_row, 0)
        pltpu.sync_copy(r0, out_h.at[pl.ds(wid * BPW, BPW)])

    return body(i0, i1, i2, i3, i4, t0, t1, t2, t3, rem)


def _gelu_exact(x):
    return 0.5 * x * (1.0 + lax.erf(x * (1.0 / math.sqrt(2.0))))


def _mlp_tc(xa, xb, w0, b0, w1, b1, w2, b2):
    blk = 2048

    def body(a_ref, b_ref, w0_ref, b0_ref, w1_ref, b1_ref, w2_ref, b2_ref,
             o_ref):
        h = a_ref[...][:, :DIM] + b_ref[...]
        h = _gelu_exact(jnp.dot(h, w0_ref[...],
                                preferred_element_type=jnp.float32) + b0_ref[...])
        h = _gelu_exact(jnp.dot(h, w1_ref[...],
                                preferred_element_type=jnp.float32) + b1_ref[...])
        o = jnp.dot(h, w2_ref[...],
                    preferred_element_type=jnp.float32) + b2_ref[...]
        o_ref[...] = o.T

    aspec = pl.BlockSpec((blk, 128), lambda i: (i, 0))
    xspec = pl.BlockSpec((blk, DIM), lambda i: (i, 0))
    wspec = pl.BlockSpec((DIM, DIM), lambda i: (0, 0))
    bspec = pl.BlockSpec((1, DIM), lambda i: (0, 0))
    return pl.pallas_call(
        body,
        grid=(BATCH // blk,),
        in_specs=[aspec, xspec, wspec, bspec, wspec, bspec, wspec, bspec],
        out_specs=pl.BlockSpec((DIM, blk), lambda i: (0, i)),
        out_shape=jax.ShapeDtypeStruct((DIM, BATCH), jnp.float32),
    )(xa, xb, w0, b0.reshape(1, DIM), w1, b1.reshape(1, DIM),
      w2, b2.reshape(1, DIM))


def kernel(idx_origin, idx_dest, idx_carrier, idx_tail_num, idx_flight_num,
           emb_origin, emb_dest, emb_carrier, emb_tail_num, emb_flight_num,
           W0, b0, W1, b1, W2, b2):
    xt = _sc_tail_gather(idx_tail_num, emb_tail_num.T)
    rem = jnp.concatenate(
        [jnp.zeros((1, DIM), jnp.float32), emb_tail_num[TAIL_MAIN:, :],
         jnp.zeros((REM_ROWS - 1 - N_REM, DIM), jnp.float32)], axis=0)
    xs = _sc_small_gather(idx_origin, idx_dest, idx_carrier, idx_flight_num,
                          idx_tail_num, emb_origin, emb_dest, emb_carrier,
                          emb_flight_num, rem)
    return _mlp_tc(xt, xs, W0, b0, W1, b1, W2, b2).T


# speedup vs baseline: 2.9428x; 1.0283x over previous
import functools
import math

import jax
import jax.numpy as jnp
from jax import lax
from jax.experimental import pallas as pl
from jax.experimental.pallas import tpu as pltpu
from jax.experimental.pallas import tpu_sc as plsc

DIM = 32
BATCH = 16384
NUM_FIELDS = 5
SCALE = 1.0 / math.sqrt(NUM_FIELDS)

NC = 2
NS = 16
NW = NC * NS
BPW = BATCH // NW
CHUNK = 128
NCH = BPW // CHUNK

V_TAIL = 1000000
TAIL_MAIN = (V_TAIL // 128) * 128
RING = 12
N_REM = V_TAIL - TAIL_MAIN
REM_ROWS = 1024
N_PAD_ROWS = REM_ROWS - 1 - N_REM


def _sc_tail_gather(idx_tail, tail_t):
    mesh = plsc.VectorSubcoreMesh(core_axis_name="c", subcore_axis_name="s")

    @functools.partial(
        pl.kernel,
        mesh=mesh,
        out_type=jax.ShapeDtypeStruct((BATCH, 128), jnp.float32),
        scratch_types=[
            pltpu.VMEM((BPW,), jnp.int32),
            pltpu.VMEM((RING, DIM, 128), jnp.float32),
            pltpu.VMEM((BPW, 128), jnp.float32),
            pltpu.SemaphoreType.DMA,
        ],
        compiler_params=pltpu.CompilerParams(use_tc_tiling_on_sc=True,
                                             needs_layout_passes=False),
    )
    def body(idx_h, tab_h, out_h, idx_v, blk, acc, sem):
        wid = lax.axis_index("s") * NC + lax.axis_index("c")
        base = wid * BPW
        pltpu.sync_copy(idx_h.at[pl.ds(base, BPW)], idx_v)
        rows16 = lax.iota(jnp.int32, 16)

        def fire(r, slot):
            c = jnp.where(r < TAIL_MAIN, r, 0) >> 7
            start = pl.multiple_of(c * 128, 128)
            pltpu.async_copy(tab_h.at[:, pl.ds(start, 128)], blk.at[slot], sem)

        def drain(slot):
            pltpu.make_async_copy(tab_h.at[:, pl.ds(0, 128)],
                                  blk.at[slot], sem).wait()

        def extract(b, r, slot):
            lane = jnp.where(r < TAIL_MAIN, r, 0) & 127
            w = jnp.where(r < TAIL_MAIN, SCALE, 0.0).astype(jnp.float32)
            lvec = jnp.full((16,), lane, jnp.int32)
            svec = jnp.full((16,), slot, jnp.int32)
            lo = plsc.load_gather(blk, [svec, rows16, lvec])
            hi = plsc.load_gather(blk, [svec, rows16 + 16, lvec])
            acc[b, pl.ds(0, 16)] = lo * w
            acc[b, pl.ds(16, 16)] = hi * w

        v0 = idx_v[pl.ds(0, 16)]
        for j in range(RING):
            fire(v0[j], j)

        def chunk_body(ci, _):
            v = idx_v[pl.ds(ci * 16, 16)]
            nxt = jnp.where(ci + 1 < BPW // 16, ci + 1, 0)
            vn = idx_v[pl.ds(nxt * 16, 16)]
            for j in range(16):
                b = ci * 16 + j
                sl = b % RING
                drain(sl)
                extract(b, v[j], sl)
                ahead = v[j + RING] if j + RING < 16 else vn[j + RING - 16]
                @pl.when(b + RING < BPW)
                def _fire_ahead(ahead=ahead, b=b):
                    fire(ahead, (b + RING) % RING)
            return _

        lax.fori_loop(0, BPW // 16, chunk_body, 0)
        pltpu.sync_copy(acc, out_h.at[pl.ds(base, BPW)])

    return body(idx_tail, tail_t)


def _sc_small_gather(i0, i1, i2, i3, i4, t0, t1, t2, t3, rem):
    mesh = plsc.VectorSubcoreMesh(core_axis_name="c", subcore_axis_name="s")

    @functools.partial(
        pl.kernel,
        mesh=mesh,
        out_type=jax.ShapeDtypeStruct((BATCH, DIM), jnp.float32),
        scratch_types=(
            [pltpu.VMEM((BPW,), jnp.int32) for _ in range(NUM_FIELDS)]
            + [pltpu.VMEM((BPW, DIM), jnp.float32) for _ in range(NUM_FIELDS)]
            + [pltpu.SemaphoreType.DMA]
        ),
        compiler_params=pltpu.CompilerParams(use_tc_tiling_on_sc=False),
    )
    def body(i0h, i1h, i2h, i3h, i4h, t0h, t1h, t2h, t3h, remh, out_h,
             x0, x1, x2, x3, x4, r0, r1, r2, r3, r4, sem):
        wid = lax.axis_index("s") * NC + lax.axis_index("c")
        base = wid * BPW
        idx_refs = (x0, x1, x2, x3, x4)
        row_refs = (r0, r1, r2, r3, r4)
        for ih, xv in zip((i0h, i1h, i2h, i3h, i4h), idx_refs):
            pltpu.sync_copy(ih.at[pl.ds(base, BPW)], xv)

        iota16 = lax.iota(jnp.int32, 16)

        def rem_body(k, carry):
            off = k * 16
            pos = base + off + iota16
            v = x4[pl.ds(off, 16)]
            spread = 1 + N_REM + (pos & 511)
            x4[pl.ds(off, 16)] = jnp.where(
                v >= TAIL_MAIN, v - TAIL_MAIN + 1, spread)
            return carry

        lax.fori_loop(0, BPW // 16, rem_body, 0)

        copies = []
        for th, xv, rv in zip((t0h, t1h, t2h, t3h, remh), idx_refs, row_refs):
            copies.append(pltpu.async_copy(th.at[xv], rv, sem))
        for c in copies:
            c.wait()

        def acc_row(r, carry):
            for h in range(DIM // 16):
                sl = pl.ds(h * 16, 16)
                r0[r, sl] = (r0[r, sl] + r1[r, sl] + r2[r, sl] + r3[r, sl]
                             + r4[r, sl]) * SCALE
            return carry

        lax.fori_loop(0, BPW, acc_row, 0)
        pltpu.sync_copy(r0, out_h.at[pl.ds(wid * BPW, BPW)])

    return body(i0, i1, i2, i3, i4, t0, t1, t2, t3, rem)


def _gelu_exact(x):
    return 0.5 * x * (1.0 + lax.erf(x * (1.0 / math.sqrt(2.0))))


def _mlp_tc(xa, xb, w0, b0, w1, b1, w2, b2):
    blk = 2048

    def body(a_ref, b_ref, w0_ref, b0_ref, w1_ref, b1_ref, w2_ref, b2_ref,
             o_ref):
        h = a_ref[...][:, :DIM] + b_ref[...]
        h = _gelu_exact(jnp.dot(h, w0_ref[...],
                                preferred_element_type=jnp.float32) + b0_ref[...])
        h = _gelu_exact(jnp.dot(h, w1_ref[...],
                                preferred_element_type=jnp.float32) + b1_ref[...])
        o = jnp.dot(h, w2_ref[...],
                    preferred_element_type=jnp.float32) + b2_ref[...]
        o_ref[...] = o.T

    aspec = pl.BlockSpec((blk, 128), lambda i: (i, 0))
    xspec = pl.BlockSpec((blk, DIM), lambda i: (i, 0))
    wspec = pl.BlockSpec((DIM, DIM), lambda i: (0, 0))
    bspec = pl.BlockSpec((1, DIM), lambda i: (0, 0))
    return pl.pallas_call(
        body,
        grid=(BATCH // blk,),
        in_specs=[aspec, xspec, wspec, bspec, wspec, bspec, wspec, bspec],
        out_specs=pl.BlockSpec((DIM, blk), lambda i: (0, i)),
        out_shape=jax.ShapeDtypeStruct((DIM, BATCH), jnp.float32),
    )(xa, xb, w0, b0.reshape(1, DIM), w1, b1.reshape(1, DIM),
      w2, b2.reshape(1, DIM))


def kernel(idx_origin, idx_dest, idx_carrier, idx_tail_num, idx_flight_num,
           emb_origin, emb_dest, emb_carrier, emb_tail_num, emb_flight_num,
           W0, b0, W1, b1, W2, b2):
    xt = _sc_tail_gather(idx_tail_num, emb_tail_num.T)
    rem = jnp.concatenate(
        [jnp.zeros((1, DIM), jnp.float32), emb_tail_num[TAIL_MAIN:, :],
         jnp.zeros((REM_ROWS - 1 - N_REM, DIM), jnp.float32)], axis=0)
    xs = _sc_small_gather(idx_origin, idx_dest, idx_carrier, idx_flight_num,
                          idx_tail_num, emb_origin, emb_dest, emb_carrier,
                          emb_flight_num, rem)
    return _mlp_tc(xt, xs, W0, b0, W1, b1, W2, b2).T


# speedup vs baseline: 2.9677x; 1.0085x over previous
import functools
import math

import jax
import jax.numpy as jnp
from jax import lax
from jax.experimental import pallas as pl
from jax.experimental.pallas import tpu as pltpu
from jax.experimental.pallas import tpu_sc as plsc

DIM = 32
BATCH = 16384
NUM_FIELDS = 5
SCALE = 1.0 / math.sqrt(NUM_FIELDS)

NC = 2
NS = 16
NW = NC * NS
BPW = BATCH // NW
CHUNK = 128
NCH = BPW // CHUNK

V_TAIL = 1000000
TAIL_MAIN = (V_TAIL // 128) * 128
RING = 12
N_REM = V_TAIL - TAIL_MAIN
REM_ROWS = 1024
N_PAD_ROWS = REM_ROWS - 1 - N_REM


def _sc_tail_gather(idx_tail, tail_t):
    mesh = plsc.VectorSubcoreMesh(core_axis_name="c", subcore_axis_name="s")

    @functools.partial(
        pl.kernel,
        mesh=mesh,
        out_type=jax.ShapeDtypeStruct((BATCH, 128), jnp.float32),
        scratch_types=[
            pltpu.VMEM((BPW,), jnp.int32),
            pltpu.VMEM((RING, DIM, 128), jnp.float32),
            pltpu.VMEM((BPW, 128), jnp.float32),
            pltpu.SemaphoreType.DMA,
        ],
        compiler_params=pltpu.CompilerParams(use_tc_tiling_on_sc=True,
                                             needs_layout_passes=False),
    )
    def body(idx_h, tab_h, out_h, idx_v, blk, acc, sem):
        wid = lax.axis_index("s") * NC + lax.axis_index("c")
        base = wid * BPW
        pltpu.sync_copy(idx_h.at[pl.ds(base, BPW)], idx_v)
        rows16 = lax.iota(jnp.int32, 16)

        def fire(r, slot):
            c = jnp.where(r < TAIL_MAIN, r, 0) >> 7
            start = pl.multiple_of(c * 128, 128)
            pltpu.async_copy(tab_h.at[:, pl.ds(start, 128)], blk.at[slot], sem)

        def drain(slot):
            pltpu.make_async_copy(tab_h.at[:, pl.ds(0, 128)],
                                  blk.at[slot], sem).wait()

        def extract(b, r, slot):
            lane = jnp.where(r < TAIL_MAIN, r, 0) & 127
            w = jnp.where(r < TAIL_MAIN, SCALE, 0.0).astype(jnp.float32)
            lvec = jnp.full((16,), lane, jnp.int32)
            svec = jnp.full((16,), slot, jnp.int32)
            lo = plsc.load_gather(blk, [svec, rows16, lvec])
            hi = plsc.load_gather(blk, [svec, rows16 + 16, lvec])
            acc[b, pl.ds(0, 16)] = lo * w
            acc[b, pl.ds(16, 16)] = hi * w

        v0 = idx_v[pl.ds(0, 16)]
        for j in range(RING):
            fire(v0[j], j)

        def chunk_body(ci, _):
            v = idx_v[pl.ds(ci * 16, 16)]
            nxt = jnp.where(ci + 1 < BPW // 16, ci + 1, 0)
            vn = idx_v[pl.ds(nxt * 16, 16)]
            for j in range(16):
                b = ci * 16 + j
                sl = b % RING
                drain(sl)
                extract(b, v[j], sl)
                ahead = v[j + RING] if j + RING < 16 else vn[j + RING - 16]
                @pl.when(b + RING < BPW)
                def _fire_ahead(ahead=ahead, b=b):
                    fire(ahead, (b + RING) % RING)
            return _

        lax.fori_loop(0, BPW // 16, chunk_body, 0)
        pltpu.sync_copy(acc, out_h.at[pl.ds(base, BPW)])

    return body(idx_tail, tail_t)


def _sc_small_gather(i0, i1, i2, i3, i4, t0, t1, t2, t3, rem):
    mesh = plsc.VectorSubcoreMesh(core_axis_name="c", subcore_axis_name="s")

    @functools.partial(
        pl.kernel,
        mesh=mesh,
        out_type=jax.ShapeDtypeStruct((BATCH, DIM), jnp.float32),
        scratch_types=(
            [pltpu.VMEM((BPW,), jnp.int32) for _ in range(NUM_FIELDS)]
            + [pltpu.VMEM((BPW, DIM), jnp.float32) for _ in range(NUM_FIELDS)]
            + [pltpu.SemaphoreType.DMA]
        ),
        compiler_params=pltpu.CompilerParams(use_tc_tiling_on_sc=False),
    )
    def body(i0h, i1h, i2h, i3h, i4h, t0h, t1h, t2h, t3h, remh, out_h,
             x0, x1, x2, x3, x4, r0, r1, r2, r3, r4, sem):
        wid = lax.axis_index("s") * NC + lax.axis_index("c")
        base = wid * BPW
        idx_refs = (x0, x1, x2, x3, x4)
        row_refs = (r0, r1, r2, r3, r4)
        icopies = [pltpu.async_copy(ih.at[pl.ds(base, BPW)], xv, sem)
                   for ih, xv in zip((i0h, i1h, i2h, i3h, i4h), idx_refs)]
        for c in icopies:
            c.wait()

        copies = [pltpu.async_copy(th.at[xv], rv, sem)
                  for th, xv, rv in zip((t0h, t1h, t2h, t3h),
                                        idx_refs[:4], row_refs[:4])]

        iota16 = lax.iota(jnp.int32, 16)

        def rem_body(k, carry):
            off = k * 16
            pos = base + off + iota16
            v = x4[pl.ds(off, 16)]
            spread = 1 + N_REM + (pos & 511)
            x4[pl.ds(off, 16)] = jnp.where(
                v >= TAIL_MAIN, v - TAIL_MAIN + 1, spread)
            return carry

        lax.fori_loop(0, BPW // 16, rem_body, 0)
        copies.append(pltpu.async_copy(remh.at[x4], r4, sem))
        for c in copies:
            c.wait()

        def acc_row(r, carry):
            for h in range(DIM // 16):
                sl = pl.ds(h * 16, 16)
                r0[r, sl] = (r0[r, sl] + r1[r, sl] + r2[r, sl] + r3[r, sl]
                             + r4[r, sl]) * SCALE
            return carry

        lax.fori_loop(0, BPW, acc_row, 0)
        pltpu.sync_copy(r0, out_h.at[pl.ds(wid * BPW, BPW)])

    return body(i0, i1, i2, i3, i4, t0, t1, t2, t3, rem)


def _gelu_exact(x):
    return 0.5 * x * (1.0 + lax.erf(x * (1.0 / math.sqrt(2.0))))


def _mlp_tc(xa, xb, w0, b0, w1, b1, w2, b2):
    blk = 2048

    def body(a_ref, b_ref, w0_ref, b0_ref, w1_ref, b1_ref, w2_ref, b2_ref,
             o_ref):
        h = a_ref[...][:, :DIM] + b_ref[...]
        h = _gelu_exact(jnp.dot(h, w0_ref[...],
                                preferred_element_type=jnp.float32) + b0_ref[...])
        h = _gelu_exact(jnp.dot(h, w1_ref[...],
                                preferred_element_type=jnp.float32) + b1_ref[...])
        o = jnp.dot(h, w2_ref[...],
                    preferred_element_type=jnp.float32) + b2_ref[...]
        o_ref[...] = o.T

    aspec = pl.BlockSpec((blk, 128), lambda i: (i, 0))
    xspec = pl.BlockSpec((blk, DIM), lambda i: (i, 0))
    wspec = pl.BlockSpec((DIM, DIM), lambda i: (0, 0))
    bspec = pl.BlockSpec((1, DIM), lambda i: (0, 0))
    return pl.pallas_call(
        body,
        grid=(BATCH // blk,),
        in_specs=[aspec, xspec, wspec, bspec, wspec, bspec, wspec, bspec],
        out_specs=pl.BlockSpec((DIM, blk), lambda i: (0, i)),
        out_shape=jax.ShapeDtypeStruct((DIM, BATCH), jnp.float32),
    )(xa, xb, w0, b0.reshape(1, DIM), w1, b1.reshape(1, DIM),
      w2, b2.reshape(1, DIM))


def kernel(idx_origin, idx_dest, idx_carrier, idx_tail_num, idx_flight_num,
           emb_origin, emb_dest, emb_carrier, emb_tail_num, emb_flight_num,
           W0, b0, W1, b1, W2, b2):
    xt = _sc_tail_gather(idx_tail_num, emb_tail_num.T)
    rem = jnp.concatenate(
        [jnp.zeros((1, DIM), jnp.float32), emb_tail_num[TAIL_MAIN:, :],
         jnp.zeros((REM_ROWS - 1 - N_REM, DIM), jnp.float32)], axis=0)
    xs = _sc_small_gather(idx_origin, idx_dest, idx_carrier, idx_flight_num,
                          idx_tail_num, emb_origin, emb_dest, emb_carrier,
                          emb_flight_num, rem)
    return _mlp_tc(xt, xs, W0, b0, W1, b1, W2, b2).T


# speedup vs baseline: 2.9688x; 1.0004x over previous
import functools
import math

import jax
import jax.numpy as jnp
from jax import lax
from jax.experimental import pallas as pl
from jax.experimental.pallas import tpu as pltpu
from jax.experimental.pallas import tpu_sc as plsc

DIM = 32
BATCH = 16384
NUM_FIELDS = 5
SCALE = 1.0 / math.sqrt(NUM_FIELDS)

NC = 2
NS = 16
NW = NC * NS
BPW = BATCH // NW
CHUNK = 128
NCH = BPW // CHUNK

V_TAIL = 1000000
TAIL_MAIN = (V_TAIL // 128) * 128
RING = 12
N_REM = V_TAIL - TAIL_MAIN
REM_ROWS = 1024
N_PAD_ROWS = REM_ROWS - 1 - N_REM


def _sc_tail_gather(idx_tail, tail_t):
    mesh = plsc.VectorSubcoreMesh(core_axis_name="c", subcore_axis_name="s")

    @functools.partial(
        pl.kernel,
        mesh=mesh,
        out_type=jax.ShapeDtypeStruct((BATCH, 64), jnp.float32),
        scratch_types=[
            pltpu.VMEM((BPW,), jnp.int32),
            pltpu.VMEM((RING, DIM, 128), jnp.float32),
            pltpu.VMEM((BPW, 64), jnp.float32),
            pltpu.SemaphoreType.DMA,
        ],
        compiler_params=pltpu.CompilerParams(use_tc_tiling_on_sc=True,
                                             needs_layout_passes=False),
    )
    def body(idx_h, tab_h, out_h, idx_v, blk, acc, sem):
        wid = lax.axis_index("s") * NC + lax.axis_index("c")
        base = wid * BPW
        pltpu.sync_copy(idx_h.at[pl.ds(base, BPW)], idx_v)
        rows16 = lax.iota(jnp.int32, 16)

        def fire(r, slot):
            c = jnp.where(r < TAIL_MAIN, r, 0) >> 7
            start = pl.multiple_of(c * 128, 128)
            pltpu.async_copy(tab_h.at[:, pl.ds(start, 128)], blk.at[slot], sem)

        def drain(slot):
            pltpu.make_async_copy(tab_h.at[:, pl.ds(0, 128)],
                                  blk.at[slot], sem).wait()

        def extract(b, r, slot):
            lane = jnp.where(r < TAIL_MAIN, r, 0) & 127
            w = jnp.where(r < TAIL_MAIN, SCALE, 0.0).astype(jnp.float32)
            lvec = jnp.full((16,), lane, jnp.int32)
            svec = jnp.full((16,), slot, jnp.int32)
            lo = plsc.load_gather(blk, [svec, rows16, lvec])
            hi = plsc.load_gather(blk, [svec, rows16 + 16, lvec])
            acc[b, pl.ds(0, 16)] = lo * w
            acc[b, pl.ds(16, 16)] = hi * w

        v0 = idx_v[pl.ds(0, 16)]
        for j in range(RING):
            fire(v0[j], j)

        def chunk_body(ci, _):
            v = idx_v[pl.ds(ci * 16, 16)]
            nxt = jnp.where(ci + 1 < BPW // 16, ci + 1, 0)
            vn = idx_v[pl.ds(nxt * 16, 16)]
            for j in range(16):
                b = ci * 16 + j
                sl = b % RING
                drain(sl)
                extract(b, v[j], sl)
                ahead = v[j + RING] if j + RING < 16 else vn[j + RING - 16]
                @pl.when(b + RING < BPW)
                def _fire_ahead(ahead=ahead, b=b):
                    fire(ahead, (b + RING) % RING)
            return _

        lax.fori_loop(0, BPW // 16, chunk_body, 0)
        pltpu.sync_copy(acc, out_h.at[pl.ds(base, BPW)])

    return body(idx_tail, tail_t)


def _sc_small_gather(i0, i1, i2, i3, i4, t0, t1, t2, t3, rem):
    mesh = plsc.VectorSubcoreMesh(core_axis_name="c", subcore_axis_name="s")

    @functools.partial(
        pl.kernel,
        mesh=mesh,
        out_type=jax.ShapeDtypeStruct((BATCH, DIM), jnp.float32),
        scratch_types=(
            [pltpu.VMEM((BPW,), jnp.int32) for _ in range(NUM_FIELDS)]
            + [pltpu.VMEM((BPW, DIM), jnp.float32) for _ in range(NUM_FIELDS)]
            + [pltpu.SemaphoreType.DMA]
        ),
        compiler_params=pltpu.CompilerParams(use_tc_tiling_on_sc=False),
    )
    def body(i0h, i1h, i2h, i3h, i4h, t0h, t1h, t2h, t3h, remh, out_h,
             x0, x1, x2, x3, x4, r0, r1, r2, r3, r4, sem):
        wid = lax.axis_index("s") * NC + lax.axis_index("c")
        base = wid * BPW
        idx_refs = (x0, x1, x2, x3, x4)
        row_refs = (r0, r1, r2, r3, r4)
        icopies = [pltpu.async_copy(ih.at[pl.ds(base, BPW)], xv, sem)
                   for ih, xv in zip((i0h, i1h, i2h, i3h, i4h), idx_refs)]
        for c in icopies:
            c.wait()

        copies = [pltpu.async_copy(th.at[xv], rv, sem)
                  for th, xv, rv in zip((t0h, t1h, t2h, t3h),
                                        idx_refs[:4], row_refs[:4])]

        iota16 = lax.iota(jnp.int32, 16)

        def rem_body(k, carry):
            off = k * 16
            pos = base + off + iota16
            v = x4[pl.ds(off, 16)]
            spread = 1 + N_REM + (pos & 511)
            x4[pl.ds(off, 16)] = jnp.where(
                v >= TAIL_MAIN, v - TAIL_MAIN + 1, spread)
            return carry

        lax.fori_loop(0, BPW // 16, rem_body, 0)
        copies.append(pltpu.async_copy(remh.at[x4], r4, sem))
        for c in copies:
            c.wait()

        def acc_row(r, carry):
            for h in range(DIM // 16):
                sl = pl.ds(h * 16, 16)
                r0[r, sl] = (r0[r, sl] + r1[r, sl] + r2[r, sl] + r3[r, sl]
                             + r4[r, sl]) * SCALE
            return carry

        lax.fori_loop(0, BPW, acc_row, 0)
        pltpu.sync_copy(r0, out_h.at[pl.ds(wid * BPW, BPW)])

    return body(i0, i1, i2, i3, i4, t0, t1, t2, t3, rem)


def _gelu_exact(x):
    return 0.5 * x * (1.0 + lax.erf(x * (1.0 / math.sqrt(2.0))))


def _mlp_tc(xa, xb, w0, b0, w1, b1, w2, b2):
    blk = 2048

    def body(a_ref, b_ref, w0_ref, b0_ref, w1_ref, b1_ref, w2_ref, b2_ref,
             o_ref):
        h = a_ref[...][:, :DIM] + b_ref[...]
        h = _gelu_exact(jnp.dot(h, w0_ref[...],
                                preferred_element_type=jnp.float32) + b0_ref[...])
        h = _gelu_exact(jnp.dot(h, w1_ref[...],
                                preferred_element_type=jnp.float32) + b1_ref[...])
        o = jnp.dot(h, w2_ref[...],
                    preferred_element_type=jnp.float32) + b2_ref[...]
        o_ref[...] = o.T

    aspec = pl.BlockSpec((blk, 64), lambda i: (i, 0))
    xspec = pl.BlockSpec((blk, DIM), lambda i: (i, 0))
    wspec = pl.BlockSpec((DIM, DIM), lambda i: (0, 0))
    bspec = pl.BlockSpec((1, DIM), lambda i: (0, 0))
    return pl.pallas_call(
        body,
        grid=(BATCH // blk,),
        in_specs=[aspec, xspec, wspec, bspec, wspec, bspec, wspec, bspec],
        out_specs=pl.BlockSpec((DIM, blk), lambda i: (0, i)),
        out_shape=jax.ShapeDtypeStruct((DIM, BATCH), jnp.float32),
    )(xa, xb, w0, b0.reshape(1, DIM), w1, b1.reshape(1, DIM),
      w2, b2.reshape(1, DIM))


def kernel(idx_origin, idx_dest, idx_carrier, idx_tail_num, idx_flight_num,
           emb_origin, emb_dest, emb_carrier, emb_tail_num, emb_flight_num,
           W0, b0, W1, b1, W2, b2):
    xt = _sc_tail_gather(idx_tail_num, emb_tail_num.T)
    rem = jnp.concatenate(
        [jnp.zeros((1, DIM), jnp.float32), emb_tail_num[TAIL_MAIN:, :],
         jnp.zeros((REM_ROWS - 1 - N_REM, DIM), jnp.float32)], axis=0)
    xs = _sc_small_gather(idx_origin, idx_dest, idx_carrier, idx_flight_num,
                          idx_tail_num, emb_origin, emb_dest, emb_carrier,
                          emb_flight_num, rem)
    return _mlp_tc(xt, xs, W0, b0, W1, b1, W2, b2).T


# speedup vs baseline: 2.9796x; 1.0036x over previous
import functools
import math

import jax
import jax.numpy as jnp
from jax import lax
from jax.experimental import pallas as pl
from jax.experimental.pallas import tpu as pltpu
from jax.experimental.pallas import tpu_sc as plsc

DIM = 32
BATCH = 16384
NUM_FIELDS = 5
SCALE = 1.0 / math.sqrt(NUM_FIELDS)

NC = 2
NS = 16
NW = NC * NS
BPW = BATCH // NW
V_TAIL = 1000000
TAIL_MAIN = (V_TAIL // 128) * 128
RING = 12
N_REM = V_TAIL - TAIL_MAIN
REM_ROWS = 1024
N_PAD_ROWS = REM_ROWS - 1 - N_REM


def _sc_tail_gather(idx_tail, tail_t):
    mesh = plsc.VectorSubcoreMesh(core_axis_name="c", subcore_axis_name="s")

    @functools.partial(
        pl.kernel,
        mesh=mesh,
        out_type=jax.ShapeDtypeStruct((BATCH, 64), jnp.float32),
        scratch_types=[
            pltpu.VMEM((BPW,), jnp.int32),
            pltpu.VMEM((RING, DIM, 128), jnp.float32),
            pltpu.VMEM((BPW, 64), jnp.float32),
            pltpu.SemaphoreType.DMA,
        ],
        compiler_params=pltpu.CompilerParams(use_tc_tiling_on_sc=True,
                                             needs_layout_passes=False),
    )
    def body(idx_h, tab_h, out_h, idx_v, blk, acc, sem):
        wid = lax.axis_index("s") * NC + lax.axis_index("c")
        base = wid * BPW
        pltpu.sync_copy(idx_h.at[pl.ds(base, BPW)], idx_v)
        rows16 = lax.iota(jnp.int32, 16)

        def fire(r, slot):
            c = jnp.where(r < TAIL_MAIN, r, 0) >> 7
            start = pl.multiple_of(c * 128, 128)
            pltpu.async_copy(tab_h.at[:, pl.ds(start, 128)], blk.at[slot], sem)

        def drain(slot):
            pltpu.make_async_copy(tab_h.at[:, pl.ds(0, 128)],
                                  blk.at[slot], sem).wait()

        def extract(b, r, slot):
            lane = jnp.where(r < TAIL_MAIN, r, 0) & 127
            w = jnp.where(r < TAIL_MAIN, SCALE, 0.0).astype(jnp.float32)
            lvec = jnp.full((16,), lane, jnp.int32)
            svec = jnp.full((16,), slot, jnp.int32)
            lo = plsc.load_gather(blk, [svec, rows16, lvec])
            hi = plsc.load_gather(blk, [svec, rows16 + 16, lvec])
            acc[b, pl.ds(0, 16)] = lo * w
            acc[b, pl.ds(16, 16)] = hi * w

        v0 = idx_v[pl.ds(0, 16)]
        for j in range(RING):
            fire(v0[j], j)

        def chunk_body(ci, _):
            v = idx_v[pl.ds(ci * 16, 16)]
            nxt = jnp.where(ci + 1 < BPW // 16, ci + 1, 0)
            vn = idx_v[pl.ds(nxt * 16, 16)]
            for j in range(16):
                b = ci * 16 + j
                sl = b % RING
                drain(sl)
                extract(b, v[j], sl)
                ahead = v[j + RING] if j + RING < 16 else vn[j + RING - 16]
                @pl.when(b + RING < BPW)
                def _fire_ahead(ahead=ahead, b=b):
                    fire(ahead, (b + RING) % RING)
            return _

        lax.fori_loop(0, BPW // 16, chunk_body, 0)
        pltpu.sync_copy(acc, out_h.at[pl.ds(base, BPW)])

    return body(idx_tail, tail_t)


def _sc_small_gather(i0, i1, i2, i3, i4, t0, t1, t2, t3, rem):
    mesh = plsc.VectorSubcoreMesh(core_axis_name="c", subcore_axis_name="s")

    @functools.partial(
        pl.kernel,
        mesh=mesh,
        out_type=jax.ShapeDtypeStruct((BATCH, DIM), jnp.float32),
        scratch_types=(
            [pltpu.VMEM((BPW,), jnp.int32) for _ in range(NUM_FIELDS)]
            + [pltpu.VMEM((BPW, DIM), jnp.float32) for _ in range(NUM_FIELDS)]
            + [pltpu.SemaphoreType.DMA]
        ),
        compiler_params=pltpu.CompilerParams(use_tc_tiling_on_sc=False),
    )
    def body(i0h, i1h, i2h, i3h, i4h, t0h, t1h, t2h, t3h, remh, out_h,
             x0, x1, x2, x3, x4, r0, r1, r2, r3, r4, sem):
        wid = lax.axis_index("s") * NC + lax.axis_index("c")
        base = wid * BPW
        idx_refs = (x0, x1, x2, x3, x4)
        row_refs = (r0, r1, r2, r3, r4)
        icopies = [pltpu.async_copy(ih.at[pl.ds(base, BPW)], xv, sem)
                   for ih, xv in zip((i0h, i1h, i2h, i3h, i4h), idx_refs)]
        for c in icopies:
            c.wait()

        copies = [pltpu.async_copy(th.at[xv], rv, sem)
                  for th, xv, rv in zip((t0h, t1h, t2h, t3h),
                                        idx_refs[:4], row_refs[:4])]

        iota16 = lax.iota(jnp.int32, 16)

        def rem_body(k, carry):
            off = k * 16
            pos = base + off + iota16
            v = x4[pl.ds(off, 16)]
            spread = 1 + N_REM + (pos & 511)
            x4[pl.ds(off, 16)] = jnp.where(
                v >= TAIL_MAIN, v - TAIL_MAIN + 1, spread)
            return carry

        lax.fori_loop(0, BPW // 16, rem_body, 0)
        copies.append(pltpu.async_copy(remh.at[x4], r4, sem))
        for c in copies:
            c.wait()

        def acc_row(r, carry):
            for h in range(DIM // 16):
                sl = pl.ds(h * 16, 16)
                r0[r, sl] = (r0[r, sl] + r1[r, sl] + r2[r, sl] + r3[r, sl]
                             + r4[r, sl]) * SCALE
            return carry

        lax.fori_loop(0, BPW, acc_row, 0)
        pltpu.sync_copy(r0, out_h.at[pl.ds(wid * BPW, BPW)])

    return body(i0, i1, i2, i3, i4, t0, t1, t2, t3, rem)


def _gelu_exact(x):
    return 0.5 * x * (1.0 + lax.erf(x * (1.0 / math.sqrt(2.0))))


def _mlp_tc(xa, xb, w0, b0, w1, b1, w2, b2):
    blk = 2048

    def body(a_ref, b_ref, w0_ref, b0_ref, w1_ref, b1_ref, w2_ref, b2_ref,
             o_ref):
        h = a_ref[...][:, :DIM] + b_ref[...]
        h = _gelu_exact(jnp.dot(h, w0_ref[...],
                                preferred_element_type=jnp.float32) + b0_ref[...])
        h = _gelu_exact(jnp.dot(h, w1_ref[...],
                                preferred_element_type=jnp.float32) + b1_ref[...])
        o = jnp.dot(h, w2_ref[...],
                    preferred_element_type=jnp.float32) + b2_ref[...]
        o_ref[...] = o.T

    aspec = pl.BlockSpec((blk, 64), lambda i: (i, 0))
    xspec = pl.BlockSpec((blk, DIM), lambda i: (i, 0))
    wspec = pl.BlockSpec((DIM, DIM), lambda i: (0, 0))
    bspec = pl.BlockSpec((1, DIM), lambda i: (0, 0))
    return pl.pallas_call(
        body,
        grid=(BATCH // blk,),
        in_specs=[aspec, xspec, wspec, bspec, wspec, bspec, wspec, bspec],
        out_specs=pl.BlockSpec((DIM, blk), lambda i: (0, i)),
        out_shape=jax.ShapeDtypeStruct((DIM, BATCH), jnp.float32),
    )(xa, xb, w0, b0.reshape(1, DIM), w1, b1.reshape(1, DIM),
      w2, b2.reshape(1, DIM))


def kernel(idx_origin, idx_dest, idx_carrier, idx_tail_num, idx_flight_num,
           emb_origin, emb_dest, emb_carrier, emb_tail_num, emb_flight_num,
           W0, b0, W1, b1, W2, b2):
    xt = _sc_tail_gather(idx_tail_num, emb_tail_num.T)
    rem = jnp.concatenate(
        [jnp.zeros((1, DIM), jnp.float32), emb_tail_num[TAIL_MAIN:, :],
         jnp.zeros((REM_ROWS - 1 - N_REM, DIM), jnp.float32)], axis=0)
    xs = _sc_small_gather(idx_origin, idx_dest, idx_carrier, idx_flight_num,
                          idx_tail_num, emb_origin, emb_dest, emb_carrier,
                          emb_flight_num, rem)
    return _mlp_tc(xt, xs, W0, b0, W1, b1, W2, b2).T


# speedup vs baseline: 2.9944x; 1.0050x over previous
import functools
import math

import jax
import jax.numpy as jnp
from jax import lax
from jax.experimental import pallas as pl
from jax.experimental.pallas import tpu as pltpu
from jax.experimental.pallas import tpu_sc as plsc

DIM = 32
BATCH = 16384
NUM_FIELDS = 5
SCALE = 1.0 / math.sqrt(NUM_FIELDS)

NC = 2
NS = 16
NW = NC * NS
BPW = BATCH // NW
V_TAIL = 1000000
TAIL_MAIN = (V_TAIL // 128) * 128
RING = 12
N_REM = V_TAIL - TAIL_MAIN
REM_ROWS = 1024
N_PAD_ROWS = REM_ROWS - 1 - N_REM


def _sc_tail_gather(idx_tail, tail_t):
    mesh = plsc.VectorSubcoreMesh(core_axis_name="c", subcore_axis_name="s")

    @functools.partial(
        pl.kernel,
        mesh=mesh,
        out_type=jax.ShapeDtypeStruct((BATCH, 64), jnp.float32),
        scratch_types=[
            pltpu.VMEM((BPW,), jnp.int32),
            pltpu.VMEM((RING, DIM, 128), jnp.float32),
            pltpu.VMEM((BPW, 64), jnp.float32),
            pltpu.SemaphoreType.DMA,
        ],
        compiler_params=pltpu.CompilerParams(use_tc_tiling_on_sc=True,
                                             needs_layout_passes=False),
    )
    def body(idx_h, tab_h, out_h, idx_v, blk, acc, sem):
        wid = lax.axis_index("s") * NC + lax.axis_index("c")
        base = wid * BPW
        pltpu.sync_copy(idx_h.at[pl.ds(base, BPW)], idx_v)
        rows16 = lax.iota(jnp.int32, 16)

        def fire(r, slot):
            c = jnp.where(r < TAIL_MAIN, r, 0) >> 7
            start = pl.multiple_of(c * 128, 128)
            pltpu.async_copy(tab_h.at[:, pl.ds(start, 128)], blk.at[slot], sem)

        def drain(slot):
            pltpu.make_async_copy(tab_h.at[:, pl.ds(0, 128)],
                                  blk.at[slot], sem).wait()

        def extract(b, r, slot):
            lane = jnp.where(r < TAIL_MAIN, r, 0) & 127
            w = jnp.where(r < TAIL_MAIN, SCALE, 0.0).astype(jnp.float32)
            lvec = jnp.full((16,), lane, jnp.int32)
            svec = jnp.full((16,), slot, jnp.int32)
            lo = plsc.load_gather(blk, [svec, rows16, lvec])
            hi = plsc.load_gather(blk, [svec, rows16 + 16, lvec])
            acc[b, pl.ds(0, 16)] = lo * w
            acc[b, pl.ds(16, 16)] = hi * w

        v0 = idx_v[pl.ds(0, 16)]
        for j in range(RING):
            fire(v0[j], j)

        def chunk_body(ci, _):
            v = idx_v[pl.ds(ci * 16, 16)]
            nxt = jnp.where(ci + 1 < BPW // 16, ci + 1, 0)
            vn = idx_v[pl.ds(nxt * 16, 16)]
            for j in range(16):
                b = ci * 16 + j
                sl = b % RING
                drain(sl)
                extract(b, v[j], sl)
                ahead = v[j + RING] if j + RING < 16 else vn[j + RING - 16]
                @pl.when(b + RING < BPW)
                def _fire_ahead(ahead=ahead, b=b):
                    fire(ahead, (b + RING) % RING)
            return _

        lax.fori_loop(0, BPW // 16, chunk_body, 0)
        pltpu.sync_copy(acc, out_h.at[pl.ds(base, BPW)])

    return body(idx_tail, tail_t)


def _sc_small_gather(i0, i1, i2, i3, i4, t0, t1, t2, t3, rem):
    mesh = plsc.VectorSubcoreMesh(core_axis_name="c", subcore_axis_name="s")

    @functools.partial(
        pl.kernel,
        mesh=mesh,
        out_type=jax.ShapeDtypeStruct((BATCH, DIM), jnp.float32),
        scratch_types=(
            [pltpu.VMEM((BPW,), jnp.int32) for _ in range(NUM_FIELDS)]
            + [pltpu.VMEM((BPW, DIM), jnp.float32) for _ in range(NUM_FIELDS)]
            + [pltpu.SemaphoreType.DMA]
        ),
        compiler_params=pltpu.CompilerParams(use_tc_tiling_on_sc=False),
    )
    def body(i0h, i1h, i2h, i3h, i4h, t0h, t1h, t2h, t3h, remh, out_h,
             x0, x1, x2, x3, x4, r0, r1, r2, r3, r4, sem):
        wid = lax.axis_index("s") * NC + lax.axis_index("c")
        base = wid * BPW
        idx_refs = (x0, x1, x2, x3, x4)
        row_refs = (r0, r1, r2, r3, r4)
        icopies = [pltpu.async_copy(ih.at[pl.ds(base, BPW)], xv, sem)
                   for ih, xv in zip((i0h, i1h, i2h, i3h, i4h), idx_refs)]
        for c in icopies:
            c.wait()

        copies = [pltpu.async_copy(th.at[xv], rv, sem)
                  for th, xv, rv in zip((t0h, t1h, t2h, t3h),
                                        idx_refs[:4], row_refs[:4])]

        iota16 = lax.iota(jnp.int32, 16)

        def rem_body(k, carry):
            off = k * 16
            pos = base + off + iota16
            v = x4[pl.ds(off, 16)]
            spread = 1 + N_REM + (pos & 511)
            x4[pl.ds(off, 16)] = jnp.where(
                v >= TAIL_MAIN, v - TAIL_MAIN + 1, spread)
            return carry

        lax.fori_loop(0, BPW // 16, rem_body, 0)
        copies.append(pltpu.async_copy(remh.at[x4], r4, sem))
        for c in copies:
            c.wait()

        def acc_row(r4i, carry):
            for u in range(4):
                r = r4i * 4 + u
                for h in range(DIM // 16):
                    sl = pl.ds(h * 16, 16)
                    r0[r, sl] = (r0[r, sl] + r1[r, sl] + r2[r, sl]
                                 + r3[r, sl] + r4[r, sl]) * SCALE
            return carry

        lax.fori_loop(0, BPW // 4, acc_row, 0)
        pltpu.sync_copy(r0, out_h.at[pl.ds(wid * BPW, BPW)])

    return body(i0, i1, i2, i3, i4, t0, t1, t2, t3, rem)


def _gelu_exact(x):
    return 0.5 * x * (1.0 + lax.erf(x * (1.0 / math.sqrt(2.0))))


def _mlp_tc(xa, xb, w0, b0, w1, b1, w2, b2):
    blk = 2048

    def body(a_ref, b_ref, w0_ref, b0_ref, w1_ref, b1_ref, w2_ref, b2_ref,
             o_ref):
        h = a_ref[...][:, :DIM] + b_ref[...]
        h = _gelu_exact(jnp.dot(h, w0_ref[...],
                                preferred_element_type=jnp.float32) + b0_ref[...])
        h = _gelu_exact(jnp.dot(h, w1_ref[...],
                                preferred_element_type=jnp.float32) + b1_ref[...])
        o = jnp.dot(h, w2_ref[...],
                    preferred_element_type=jnp.float32) + b2_ref[...]
        o_ref[...] = o.T

    aspec = pl.BlockSpec((blk, 64), lambda i: (i, 0))
    xspec = pl.BlockSpec((blk, DIM), lambda i: (i, 0))
    wspec = pl.BlockSpec((DIM, DIM), lambda i: (0, 0))
    bspec = pl.BlockSpec((1, DIM), lambda i: (0, 0))
    return pl.pallas_call(
        body,
        grid=(BATCH // blk,),
        in_specs=[aspec, xspec, wspec, bspec, wspec, bspec, wspec, bspec],
        out_specs=pl.BlockSpec((DIM, blk), lambda i: (0, i)),
        out_shape=jax.ShapeDtypeStruct((DIM, BATCH), jnp.float32),
    )(xa, xb, w0, b0.reshape(1, DIM), w1, b1.reshape(1, DIM),
      w2, b2.reshape(1, DIM))


def kernel(idx_origin, idx_dest, idx_carrier, idx_tail_num, idx_flight_num,
           emb_origin, emb_dest, emb_carrier, emb_tail_num, emb_flight_num,
           W0, b0, W1, b1, W2, b2):
    xt = _sc_tail_gather(idx_tail_num, emb_tail_num.T)
    rem = jnp.concatenate(
        [jnp.zeros((1, DIM), jnp.float32), emb_tail_num[TAIL_MAIN:, :],
         jnp.zeros((REM_ROWS - 1 - N_REM, DIM), jnp.float32)], axis=0)
    xs = _sc_small_gather(idx_origin, idx_dest, idx_carrier, idx_flight_num,
                          idx_tail_num, emb_origin, emb_dest, emb_carrier,
                          emb_flight_num, rem)
    return _mlp_tc(xt, xs, W0, b0, W1, b1, W2, b2).T
